# baseline (speedup 1.0000x reference)
import numpy as np
import jax
import jax.numpy as jnp
from jax import lax
from jax.experimental import pallas as pl
from jax.experimental.pallas import tpu as pltpu
from jax.experimental.pallas import tpu_sc as plsc

_N = 512
_B = 256
_NOUT = _N * (_N - 1) // 2
_NCORE = 2
_NSUB = 16
_CH = _NOUT // _NSUB
_NVEC = _CH // 16
_G = 128
_UPAD = 128
_UPB = _N * _N // _G


def _build_tables():
    r, c = np.tril_indices(_N, k=-1)
    w = r.astype(np.int64) * _N + c.astype(np.int64)
    units = np.zeros((_NSUB, _UPAD), np.int32)
    lidx = np.zeros((_NSUB, _CH), np.int32)
    for ci in range(_NSUB):
        ww = w[ci * _CH:(ci + 1) * _CH]
        u = np.unique(ww // _G)
        assert len(u) <= _UPAD
        units[ci, :len(u)] = u
        units[ci, len(u):] = u[-1]
        pos = np.searchsorted(u, ww // _G)
        lidx[ci] = (pos * _G + ww % _G).astype(np.int32)
    return units.reshape(-1), lidx.reshape(-1)


_UNITS_NP, _LIDX_NP = _build_tables()


def _tril_body(xr, units, lidx, out, ubuf, libuf, staged, obuf, gsem):
    chunk = lax.axis_index("s")
    half = lax.axis_index("c")
    ubase = pl.multiple_of(chunk * _UPAD, 8)
    pltpu.sync_copy(units.at[pl.ds(ubase, _UPAD)], ubuf)
    lbase = pl.multiple_of(chunk * _CH, 8)
    pltpu.sync_copy(lidx.at[pl.ds(lbase, _CH)], libuf)

    inc0 = jnp.full((16,), half * _UPB, jnp.int32)
    for k in range(_UPAD // 16):
        ubuf[pl.ds(16 * k, 16)] = ubuf[pl.ds(16 * k, 16)] + inc0
    step = jnp.full((16,), _NCORE * _UPB, jnp.int32)

    def body(t, carry):
        b = half + _NCORE * t
        pltpu.async_copy(xr.at[ubuf], staged, gsem).wait()

        def gbody(i, _):
            iv = libuf[pl.ds(16 * i, 16)]
            row = lax.shift_right_logical(iv, 7)
            col = lax.bitwise_and(iv, _G - 1)
            obuf[pl.ds(16 * i, 16)] = plsc.load_gather(staged, [row, col])
            return 0

        lax.fori_loop(0, _NVEC, gbody, 0)
        obase = pl.multiple_of(b * _NOUT + chunk * _CH, 8)
        pltpu.sync_copy(obuf, out.at[pl.ds(obase, _CH)])

        for k in range(_UPAD // 16):
            ubuf[pl.ds(16 * k, 16)] = ubuf[pl.ds(16 * k, 16)] + step
        return carry

    lax.fori_loop(0, _B // _NCORE, body, 0)


@jax.jit
def _tril_gather(xr, units, lidx):
    info = plsc.get_sparse_core_info()
    assert info.num_cores == _NCORE and info.num_subcores == _NSUB
    mesh = plsc.VectorSubcoreMesh(core_axis_name="c", subcore_axis_name="s")
    return pl.kernel(
        _tril_body,
        mesh=mesh,
        out_type=jax.ShapeDtypeStruct((_B * _NOUT,), jnp.float32),
        scratch_types=[
            pltpu.VMEM((_UPAD,), jnp.int32),
            pltpu.VMEM((_CH,), jnp.int32),
            pltpu.VMEM((_UPAD, _G), jnp.float32),
            pltpu.VMEM((_CH,), jnp.float32),
            pltpu.SemaphoreType.DMA,
        ],
        compiler_params=pltpu.CompilerParams(needs_layout_passes=False),
    )(xr, units, lidx)


def kernel(X):
    xr = X.reshape(_B * _UPB, _G)
    flat = _tril_gather(xr, jnp.asarray(_UNITS_NP), jnp.asarray(_LIDX_NP))
    return flat.reshape(_B, _NOUT)


# speedup vs baseline: 1.7001x; 1.7001x over previous
import numpy as np
import jax
import jax.numpy as jnp
from jax import lax
from jax.experimental import pallas as pl
from jax.experimental.pallas import tpu as pltpu
from jax.experimental.pallas import tpu_sc as plsc

_N = 512
_B = 256
_NOUT = _N * (_N - 1) // 2
_NCORE = 2
_NSUB = 16
_CH = _NOUT // _NSUB
_NVEC = _CH // 16
_G = 128
_UPAD = 128
_UPB = _N * _N // _G


def _build_tables():
    r, c = np.tril_indices(_N, k=-1)
    w = r.astype(np.int64) * _N + c.astype(np.int64)
    units = np.zeros((_NSUB, _UPAD), np.int32)
    lidx = np.zeros((_NSUB, _CH), np.int32)
    for ci in range(_NSUB):
        ww = w[ci * _CH:(ci + 1) * _CH]
        u = np.unique(ww // _G)
        assert len(u) <= _UPAD
        units[ci, :len(u)] = u
        units[ci, len(u):] = u[-1]
        pos = np.searchsorted(u, ww // _G)
        lidx[ci] = (pos * _G + ww % _G).astype(np.int32)
    return units.reshape(-1), lidx.reshape(-1)


_UNITS_NP, _LIDX_NP = _build_tables()


_TT = _B // _NCORE // 2


def _tril_body(xr, units, lidx, out,
               ub0, ub1, libuf, st0, st1, ob0, ob1, gs0, gs1, os0, os1):
    chunk = lax.axis_index("s")
    half = lax.axis_index("c")
    ubase = pl.multiple_of(chunk * _UPAD, 8)
    pltpu.sync_copy(units.at[pl.ds(ubase, _UPAD)], ub0)
    pltpu.sync_copy(units.at[pl.ds(ubase, _UPAD)], ub1)
    lbase = pl.multiple_of(chunk * _CH, 8)
    pltpu.sync_copy(lidx.at[pl.ds(lbase, _CH)], libuf)

    inc0 = jnp.full((16,), half * _UPB, jnp.int32)
    inc1 = jnp.full((16,), (half + _NCORE) * _UPB, jnp.int32)
    for k in range(_UPAD // 16):
        ub0[pl.ds(16 * k, 16)] = ub0[pl.ds(16 * k, 16)] + inc0
        ub1[pl.ds(16 * k, 16)] = ub1[pl.ds(16 * k, 16)] + inc1

    pltpu.async_copy(xr.at[ub0], st0, gs0)
    pltpu.async_copy(xr.at[ub1], st1, gs1)

    step2 = jnp.full((16,), 2 * _NCORE * _UPB, jnp.int32)

    def slot(tt, t, ub, st, ob, gsem, osem):
        b = half + _NCORE * t
        obase = pl.multiple_of(b * _NOUT + chunk * _CH, 8)
        pltpu.make_async_copy(xr.at[ub], st, gsem).wait()
        for k in range(_UPAD // 16):
            ub[pl.ds(16 * k, 16)] = ub[pl.ds(16 * k, 16)] + step2

        @pl.when(tt > 0)
        def _():
            pltpu.make_async_copy(ob, out.at[pl.ds(obase, _CH)], osem).wait()

        @plsc.parallel_loop(0, _CH, step=16, unroll=8)
        def _gloop(i):
            iv = libuf[pl.ds(i, 16)]
            row = lax.shift_right_logical(iv, 7)
            col = lax.bitwise_and(iv, _G - 1)
            ob[pl.ds(i, 16)] = plsc.load_gather(st, [row, col])

        pltpu.async_copy(ob, out.at[pl.ds(obase, _CH)], osem)

        @pl.when(tt < _TT - 1)
        def _():
            pltpu.async_copy(xr.at[ub], st, gsem)

    def body(tt, carry):
        slot(tt, 2 * tt, ub0, st0, ob0, gs0, os0)
        slot(tt, 2 * tt + 1, ub1, st1, ob1, gs1, os1)
        return carry

    lax.fori_loop(0, _TT, body, 0)

    for t, ob, osem in ((2 * _TT - 2, ob0, os0), (2 * _TT - 1, ob1, os1)):
        b = half + _NCORE * t
        obase = pl.multiple_of(b * _NOUT + chunk * _CH, 8)
        pltpu.make_async_copy(ob, out.at[pl.ds(obase, _CH)], osem).wait()


@jax.jit
def _tril_gather(xr, units, lidx):
    info = plsc.get_sparse_core_info()
    assert info.num_cores == _NCORE and info.num_subcores == _NSUB
    mesh = plsc.VectorSubcoreMesh(core_axis_name="c", subcore_axis_name="s")
    return pl.kernel(
        _tril_body,
        mesh=mesh,
        out_type=jax.ShapeDtypeStruct((_B * _NOUT,), jnp.float32),
        scratch_types=[
            pltpu.VMEM((_UPAD,), jnp.int32),
            pltpu.VMEM((_UPAD,), jnp.int32),
            pltpu.VMEM((_CH,), jnp.int32),
            pltpu.VMEM((_UPAD, _G), jnp.float32),
            pltpu.VMEM((_UPAD, _G), jnp.float32),
            pltpu.VMEM((_CH,), jnp.float32),
            pltpu.VMEM((_CH,), jnp.float32),
            pltpu.SemaphoreType.DMA,
            pltpu.SemaphoreType.DMA,
            pltpu.SemaphoreType.DMA,
            pltpu.SemaphoreType.DMA,
        ],
        compiler_params=pltpu.CompilerParams(needs_layout_passes=False),
    )(xr, units, lidx)


def kernel(X):
    xr = X.reshape(_B * _UPB, _G)
    flat = _tril_gather(xr, jnp.asarray(_UNITS_NP), jnp.asarray(_LIDX_NP))
    return flat.reshape(_B, _NOUT)


# speedup vs baseline: 1.9501x; 1.1471x over previous
import numpy as np
import jax
import jax.numpy as jnp
from jax import lax
from jax.experimental import pallas as pl
from jax.experimental.pallas import tpu as pltpu
from jax.experimental.pallas import tpu_sc as plsc

_N = 512
_B = 256
_NOUT = _N * (_N - 1) // 2
_NCORE = 2
_NSUB = 16
_CH = _NOUT // _NSUB
_NVEC = _CH // 16
_G = 128
_UPAD = 128
_UPB = _N * _N // _G


def _build_tables():
    r, c = np.tril_indices(_N, k=-1)
    w = r.astype(np.int64) * _N + c.astype(np.int64)
    units = np.zeros((_NSUB, _UPAD), np.int32)
    lidx = np.zeros((_NSUB, _CH), np.int32)
    for ci in range(_NSUB):
        ww = w[ci * _CH:(ci + 1) * _CH]
        u = np.unique(ww // _G)
        assert len(u) <= _UPAD
        units[ci, :len(u)] = u
        units[ci, len(u):] = u[-1]
        pos = np.searchsorted(u, ww // _G)
        lidx[ci] = (pos * _G + ww % _G).astype(np.int32)
    return units.reshape(-1), lidx.reshape(-1)


_UNITS_NP, _LIDX_NP = _build_tables()


_NSLOT = 4
_TT = _B // _NCORE // _NSLOT


def _tril_body(xr, units, lidx, out, ubs, libuf, sts, obs, gsems, osems):
    chunk = lax.axis_index("s")
    half = lax.axis_index("c")
    ubase = pl.multiple_of(chunk * _UPAD, 8)
    lbase = pl.multiple_of(chunk * _CH, 8)
    pltpu.sync_copy(lidx.at[pl.ds(lbase, _CH)], libuf)

    for i in range(_NSLOT):
        pltpu.sync_copy(units.at[pl.ds(ubase, _UPAD)], ubs[i])
        inc = jnp.full((16,), (half + i * _NCORE) * _UPB, jnp.int32)
        for k in range(_UPAD // 16):
            ubs[i][pl.ds(16 * k, 16)] = ubs[i][pl.ds(16 * k, 16)] + inc
        pltpu.async_copy(xr.at[ubs[i]], sts[i], gsems[i])

    step_inc = jnp.full((16,), _NSLOT * _NCORE * _UPB, jnp.int32)

    def slot(tt, i, ub, st, ob, gsem, osem):
        t = _NSLOT * tt + i
        b = half + _NCORE * t
        obase = pl.multiple_of(b * _NOUT + chunk * _CH, 8)
        pltpu.make_async_copy(xr.at[ub], st, gsem).wait()
        for k in range(_UPAD // 16):
            ub[pl.ds(16 * k, 16)] = ub[pl.ds(16 * k, 16)] + step_inc

        @pl.when(tt > 0)
        def _():
            pltpu.make_async_copy(ob, out.at[pl.ds(obase, _CH)], osem).wait()

        @plsc.parallel_loop(0, _CH, step=16, unroll=8)
        def _gloop(idx):
            iv = libuf[pl.ds(idx, 16)]
            row = lax.shift_right_logical(iv, 7)
            col = lax.bitwise_and(iv, _G - 1)
            ob[pl.ds(idx, 16)] = plsc.load_gather(st, [row, col])

        pltpu.async_copy(ob, out.at[pl.ds(obase, _CH)], osem)

        @pl.when(tt < _TT - 1)
        def _():
            pltpu.async_copy(xr.at[ub], st, gsem)

    def body(tt, carry):
        for i in range(_NSLOT):
            slot(tt, i, ubs[i], sts[i], obs[i], gsems[i], osems[i])
        return carry

    lax.fori_loop(0, _TT, body, 0)

    for i in range(_NSLOT):
        b = half + _NCORE * (_NSLOT * (_TT - 1) + i)
        obase = pl.multiple_of(b * _NOUT + chunk * _CH, 8)
        pltpu.make_async_copy(obs[i], out.at[pl.ds(obase, _CH)], osems[i]).wait()


@jax.jit
def _tril_gather(xr, units, lidx):
    info = plsc.get_sparse_core_info()
    assert info.num_cores == _NCORE and info.num_subcores == _NSUB
    mesh = plsc.VectorSubcoreMesh(core_axis_name="c", subcore_axis_name="s")
    return pl.kernel(
        _tril_body,
        mesh=mesh,
        out_type=jax.ShapeDtypeStruct((_B * _NOUT,), jnp.float32),
        scratch_types=[
            [pltpu.VMEM((_UPAD,), jnp.int32)] * _NSLOT,
            pltpu.VMEM((_CH,), jnp.int32),
            [pltpu.VMEM((_UPAD, _G), jnp.float32)] * _NSLOT,
            [pltpu.VMEM((_CH,), jnp.float32)] * _NSLOT,
            [pltpu.SemaphoreType.DMA] * _NSLOT,
            [pltpu.SemaphoreType.DMA] * _NSLOT,
        ],
        compiler_params=pltpu.CompilerParams(needs_layout_passes=False),
    )(xr, units, lidx)


def kernel(X):
    xr = X.reshape(_B * _UPB, _G)
    flat = _tril_gather(xr, jnp.asarray(_UNITS_NP), jnp.asarray(_LIDX_NP))
    return flat.reshape(_B, _NOUT)


# speedup vs baseline: 5.8277x; 2.9885x over previous
import numpy as np
import jax
import jax.numpy as jnp
from jax import lax
from jax.experimental import pallas as pl
from jax.experimental.pallas import tpu as pltpu
from jax.experimental.pallas import tpu_sc as plsc

_N = 512
_B = 256
_NOUT = _N * (_N - 1) // 2
_NCORE = 2
_NSUB = 16
_CH = _NOUT // _NSUB
_GR = 16
_NROW = 2 * _GR
_NSLOT = 4
_TT = _B // _NCORE // _NSLOT

_LA = [256 * s + 120 for s in range(_NSUB)]
_OFFA = [(_GR * s) * (_GR * s - 1) // 2 for s in range(_NSUB)]
_OFFB = [(_GR * (31 - s)) * (_GR * (31 - s) - 1) // 2 for s in range(_NSUB)]


def _build_tables():
    rows = np.zeros((_NSUB, _NROW), np.int32)
    lidx = np.zeros((_NSUB, _CH), np.int32)
    for s in range(_NSUB):
        glist = list(range(_GR * s, _GR * s + _GR)) + \
                list(range(_GR * (31 - s), _GR * (31 - s) + _GR))
        rows[s] = glist
        pieces = [np.arange(r, dtype=np.int32) + 512 * q
                  for q, r in enumerate(glist)]
        li = np.concatenate(pieces)
        assert li.size == _CH and li.size - _LA[s] == 8176 - _LA[s]
        lidx[s] = li
    return rows.reshape(-1), lidx.reshape(-1)


_ROWS_NP, _LIDX_NP = _build_tables()


def _tril_body(xt, rows, lidx, out, ubs, libuf, sts, obs, gsems, osems):
    sub = lax.axis_index("s")
    half = lax.axis_index("c")
    ubase = pl.multiple_of(sub * _NROW, 8)
    lbase = pl.multiple_of(sub * _CH, 8)
    pltpu.sync_copy(lidx.at[pl.ds(lbase, _CH)], libuf)

    for i in range(_NSLOT):
        pltpu.sync_copy(rows.at[pl.ds(ubase, _NROW)], ubs[i])
        inc = jnp.full((16,), (half + i * _NCORE) * _N, jnp.int32)
        for k in range(_NROW // 16):
            ubs[i][pl.ds(16 * k, 16)] = ubs[i][pl.ds(16 * k, 16)] + inc
        pltpu.async_copy(xt.at[ubs[i]], sts[i], gsems[i])

    step_inc = jnp.full((16,), _NSLOT * _NCORE * _N, jnp.int32)

    def emit_out(ob, b, osem):
        def branch(p):
            def go():
                la, offa, offb = _LA[p], _OFFA[p], _OFFB[p]
                basea = pl.multiple_of(b * _NOUT + offa, 8)
                baseb = pl.multiple_of(b * _NOUT + offb, 8)
                pltpu.async_copy(ob.at[pl.ds(0, la)],
                                 out.at[pl.ds(basea, la)], osem)
                pltpu.async_copy(ob.at[pl.ds(la, _CH - la)],
                                 out.at[pl.ds(baseb, _CH - la)], osem)
            return go
        lax.switch(sub, [branch(p) for p in range(_NSUB)])

    def slot(tt, i, ub, st, ob, gsem, osem):
        b = half + _NCORE * (_NSLOT * tt + i)
        pltpu.make_async_copy(xt.at[ub], st, gsem).wait()
        for k in range(_NROW // 16):
            ub[pl.ds(16 * k, 16)] = ub[pl.ds(16 * k, 16)] + step_inc

        @pl.when(tt > 0)
        def _():
            pltpu.make_async_copy(ob, out.at[pl.ds(0, _CH)], osem).wait()

        @plsc.parallel_loop(0, _CH, step=16, unroll=8)
        def _gloop(idx):
            iv = libuf[pl.ds(idx, 16)]
            row = lax.shift_right_logical(iv, 9)
            col = lax.bitwise_and(iv, _N - 1)
            ob[pl.ds(idx, 16)] = plsc.load_gather(st, [row, col])

        emit_out(ob, b, osem)

        @pl.when(tt < _TT - 1)
        def _():
            pltpu.async_copy(xt.at[ub], st, gsem)

    def body(tt, carry):
        for i in range(_NSLOT):
            slot(tt, i, ubs[i], sts[i], obs[i], gsems[i], osems[i])
        return carry

    lax.fori_loop(0, _TT, body, 0)

    for i in range(_NSLOT):
        pltpu.make_async_copy(obs[i], out.at[pl.ds(0, _CH)], osems[i]).wait()


@jax.jit
def _tril_gather(xt, rows, lidx):
    info = plsc.get_sparse_core_info()
    assert info.num_cores == _NCORE and info.num_subcores == _NSUB
    mesh = plsc.VectorSubcoreMesh(core_axis_name="c", subcore_axis_name="s")
    return pl.kernel(
        _tril_body,
        mesh=mesh,
        out_type=jax.ShapeDtypeStruct((_B * _NOUT,), jnp.float32),
        scratch_types=[
            [pltpu.VMEM((_NROW,), jnp.int32)] * _NSLOT,
            pltpu.VMEM((_CH,), jnp.int32),
            [pltpu.VMEM((_NROW, _N), jnp.float32)] * _NSLOT,
            [pltpu.VMEM((_CH,), jnp.float32)] * _NSLOT,
            [pltpu.SemaphoreType.DMA] * _NSLOT,
            [pltpu.SemaphoreType.DMA] * _NSLOT,
        ],
        compiler_params=pltpu.CompilerParams(needs_layout_passes=False),
    )(xt, rows, lidx)


def kernel(X):
    xt = X.reshape(_B * _N, _N)
    flat = _tril_gather(xt, jnp.asarray(_ROWS_NP), jnp.asarray(_LIDX_NP))
    return flat.reshape(_B, _NOUT)
